# baseline (speedup 1.0000x reference)
import functools

import jax
import jax.numpy as jnp
from jax import lax
from jax.experimental import pallas as pl
from jax.experimental.pallas import tpu as pltpu
from jax.experimental.pallas import tpu_sc as plsc

BATCH = 16384
F = 4096
L = 16
NW = 32
ROWS_PER_TILE = BATCH // NW
CHUNK = 2
NBUF = 4
NCHUNKS = ROWS_PER_TILE // CHUNK
NG = NCHUNKS // NBUF
JBLOCKS = F // L


def _gather_chunk(perm_v, in_v, out_v):
    @plsc.parallel_loop(0, JBLOCKS, unroll=8)
    def j_body(j):
        col0 = pl.multiple_of(j * L, L)
        idx = perm_v[pl.ds(col0, L)]
        for r in range(CHUNK):
            row_idx = jnp.full((L,), r, dtype=jnp.int32)
            vals = plsc.load_gather(in_v, [row_idx, idx])
            out_v[r, pl.ds(col0, L)] = vals


def _permute_body(in_hbm, perm_hbm, out_hbm, perm_v, in_bufs, out_bufs,
                  in_sems, out_sems):
    wid = lax.axis_index("s") * 2 + lax.axis_index("c")
    base = wid * ROWS_PER_TILE
    pltpu.sync_copy(perm_hbm, perm_v)

    def rows(c):
        return pl.ds(base + c * CHUNK, CHUNK)

    def start_in(c, b):
        pltpu.async_copy(in_hbm.at[rows(c)], in_bufs[b], in_sems[b])

    def wait_in(c, b):
        pltpu.make_async_copy(in_hbm.at[rows(c)], in_bufs[b], in_sems[b]).wait()

    def start_out(c, b):
        pltpu.async_copy(out_bufs[b], out_hbm.at[rows(c)], out_sems[b])

    def wait_out(c, b):
        pltpu.make_async_copy(out_bufs[b], out_hbm.at[rows(c)], out_sems[b]).wait()

    for b in range(NBUF):
        start_in(b, b)
    for b in range(NBUF):
        wait_in(b, b)
        _gather_chunk(perm_v, in_bufs[b], out_bufs[b])
        start_out(b, b)
        start_in(b + NBUF, b)

    def group_body(g, _):
        for b in range(NBUF):
            c = NBUF * g + b
            wait_in(c, b)
            wait_out(c - NBUF, b)
            _gather_chunk(perm_v, in_bufs[b], out_bufs[b])
            start_out(c, b)
            c2 = jnp.minimum(c + NBUF, NCHUNKS - 1)
            start_in(c2, b)
        return 0

    lax.fori_loop(1, NG, group_body, 0)

    for b in range(NBUF):
        wait_in(NCHUNKS - 1, b)
        wait_out(NCHUNKS - NBUF + b, b)


@functools.partial(
    pl.kernel,
    mesh=plsc.VectorSubcoreMesh(core_axis_name="c", subcore_axis_name="s"),
    out_type=jax.ShapeDtypeStruct((BATCH, F), jnp.float32),
    scratch_types=(
        [pltpu.VMEM((F,), jnp.int32)]
        + [pltpu.VMEM((CHUNK, F), jnp.float32)] * (2 * NBUF)
        + [pltpu.SemaphoreType.DMA] * (2 * NBUF)
    ),
    compiler_params=pltpu.CompilerParams(
        needs_layout_passes=False, skip_device_barrier=True
    ),
)
def _permute_kernel(in_hbm, perm_hbm, out_hbm, perm_v, *rest):
    bufs, sems = rest[: 2 * NBUF], rest[2 * NBUF:]
    _permute_body(
        in_hbm, perm_hbm, out_hbm, perm_v,
        bufs[:NBUF], bufs[NBUF:], sems[:NBUF], sems[NBUF:],
    )


def kernel(inputs, permutation):
    outputs = _permute_kernel(inputs, permutation.astype(jnp.int32))
    logabsdet = jnp.zeros((inputs.shape[0],), dtype=inputs.dtype)
    return (outputs, logabsdet)


# speedup vs baseline: 1.0013x; 1.0013x over previous
import functools

import jax
import jax.numpy as jnp
from jax import lax
from jax.experimental import pallas as pl
from jax.experimental.pallas import tpu as pltpu
from jax.experimental.pallas import tpu_sc as plsc

BATCH = 16384
F = 4096
L = 16
NW = 32
ROWS_PER_TILE = BATCH // NW
CHUNK = 2
NBUF = 4
NCHUNKS = ROWS_PER_TILE // CHUNK
NG = NCHUNKS // NBUF
JBLOCKS = F // L


def _gather_chunk(perm_v, in_v, out_v):
    @plsc.parallel_loop(0, JBLOCKS, unroll=8)
    def j_body(j):
        col0 = pl.multiple_of(j * L, L)
        idx = perm_v[pl.ds(col0, L)]
        for r in range(CHUNK):
            row_idx = jnp.full((L,), r, dtype=jnp.int32)
            vals = plsc.load_gather(in_v, [row_idx, idx])
            out_v[r, pl.ds(col0, L)] = vals


def _permute_body(in_hbm, perm_hbm, out_hbm, ld_hbm, perm_v, zero_v,
                  in_bufs, out_bufs, in_sems, out_sems):
    wid = lax.axis_index("s") * 2 + lax.axis_index("c")
    base = wid * ROWS_PER_TILE

    def rows(c):
        return pl.ds(base + c * CHUNK, CHUNK)

    def start_in(c, b):
        pltpu.async_copy(in_hbm.at[rows(c)], in_bufs[b], in_sems[b])

    def wait_in(c, b):
        pltpu.make_async_copy(in_hbm.at[rows(c)], in_bufs[b], in_sems[b]).wait()

    def start_out(c, b):
        pltpu.async_copy(out_bufs[b], out_hbm.at[rows(c)], out_sems[b])

    def wait_out(c, b):
        pltpu.make_async_copy(out_bufs[b], out_hbm.at[rows(c)], out_sems[b]).wait()

    for b in range(NBUF):
        start_in(b, b)
    pltpu.sync_copy(perm_hbm, perm_v)

    ld_per_tile = BATCH // NW
    for i in range(ld_per_tile // L):
        zero_v[pl.ds(i * L, L)] = jnp.zeros((L,), jnp.float32)
    pltpu.sync_copy(zero_v, ld_hbm.at[pl.ds(wid * ld_per_tile, ld_per_tile)])

    for b in range(NBUF):
        wait_in(b, b)
        _gather_chunk(perm_v, in_bufs[b], out_bufs[b])
        start_out(b, b)
        start_in(b + NBUF, b)

    def group_body(g, _):
        for b in range(NBUF):
            c = NBUF * g + b
            wait_in(c, b)
            wait_out(c - NBUF, b)
            _gather_chunk(perm_v, in_bufs[b], out_bufs[b])
            start_out(c, b)
            c2 = jnp.minimum(c + NBUF, NCHUNKS - 1)
            start_in(c2, b)
        return 0

    lax.fori_loop(1, NG, group_body, 0)

    for b in range(NBUF):
        wait_in(NCHUNKS - 1, b)
        wait_out(NCHUNKS - NBUF + b, b)


@functools.partial(
    pl.kernel,
    mesh=plsc.VectorSubcoreMesh(core_axis_name="c", subcore_axis_name="s"),
    out_type=(
        jax.ShapeDtypeStruct((BATCH, F), jnp.float32),
        jax.ShapeDtypeStruct((BATCH,), jnp.float32),
    ),
    scratch_types=(
        [pltpu.VMEM((F,), jnp.int32), pltpu.VMEM((BATCH // NW,), jnp.float32)]
        + [pltpu.VMEM((CHUNK, F), jnp.float32)] * (2 * NBUF)
        + [pltpu.SemaphoreType.DMA] * (2 * NBUF)
    ),
    compiler_params=pltpu.CompilerParams(
        needs_layout_passes=False, skip_device_barrier=True
    ),
)
def _permute_kernel(in_hbm, perm_hbm, out_hbm, ld_hbm, perm_v, zero_v, *rest):
    bufs, sems = rest[: 2 * NBUF], rest[2 * NBUF:]
    _permute_body(
        in_hbm, perm_hbm, out_hbm, ld_hbm, perm_v, zero_v,
        bufs[:NBUF], bufs[NBUF:], sems[:NBUF], sems[NBUF:],
    )


def kernel(inputs, permutation):
    outputs, logabsdet = _permute_kernel(inputs, permutation.astype(jnp.int32))
    return (outputs, logabsdet)
